# baseline (speedup 1.0000x reference)
import functools

import jax
import jax.numpy as jnp
from jax import lax
from jax.experimental import pallas as pl
from jax.experimental.pallas import tpu as pltpu
from jax.experimental.pallas import tpu_sc as plsc

_NF = 26
_V = 100000
_D = 32
_B = 4096

_Q = 49920
_SPLIT = ((0, _Q), (_Q, _V - _Q))

_INFO = plsc.get_sparse_core_info()
_NC = _INFO.num_cores
_NS = _INFO.num_subcores
_NW = _NC * _NS
_CPW = _NF * _D // _NW
_UNROLL = 8


def _make_kernel():
    mesh = plsc.VectorSubcoreMesh(core_axis_name="c", subcore_axis_name="s")

    @functools.partial(
        pl.kernel,
        mesh=mesh,
        out_type=jax.ShapeDtypeStruct((_NF * _D, _B), jnp.float32),
        scratch_types=[
            pltpu.VMEM((_SPLIT[0][1],), jnp.float32),
            pltpu.VMEM((_SPLIT[1][1],), jnp.float32),
            pltpu.VMEM((_B,), jnp.int32),
            pltpu.VMEM((_B,), jnp.float32),
            pltpu.SemaphoreType.DMA,
            pltpu.SemaphoreType.DMA,
        ],
        compiler_params=pltpu.CompilerParams(use_tc_tiling_on_sc=True,
                                             needs_layout_passes=False),
    )
    def col_kernel(idx_hbm, tab_hbm, out_hbm, q0, q1, idx_v, row_v,
                   s0, s1):
        bufs = (q0, q1)
        sems = (s0, s1)
        w = lax.axis_index("s") * _NC + lax.axis_index("c")
        c0 = w * _CPW
        lane = lax.iota(jnp.int32, 16)

        for q, (off, ln) in enumerate(_SPLIT):
            pltpu.async_copy(tab_hbm.at[c0 // _D, c0 % _D, pl.ds(off, ln)],
                             bufs[q], sems[q])

        def pair(j, f_prev):
            c = c0 + j
            f = c // _D
            d = c % _D

            @pl.when(f != f_prev)
            def _():
                pltpu.sync_copy(idx_hbm.at[f], idx_v)

            for q, (off, ln) in enumerate(_SPLIT):
                pltpu.make_async_copy(tab_hbm.at[f, d, pl.ds(off, ln)],
                                      bufs[q], sems[q]).wait()
                buf = bufs[q]

                def gat(i, c2, q=q, off=off, ln=ln, buf=buf):
                    for u in range(_UNROLL):
                        sl = pl.ds((i * _UNROLL + u) * 16, 16)
                        ix = idx_v[sl] - off
                        m = lax.bitcast_convert_type(ix, jnp.uint32) < ln
                        g = plsc.load_gather(buf, [ix], mask=m)
                        if q == 0:
                            row_v[sl] = jnp.where(m, g, 0.0)
                        else:
                            row_v[sl] = jnp.where(m, g, row_v[sl])
                    return c2

                lax.fori_loop(0, _B // (16 * _UNROLL), gat, 0)

                @pl.when(j < _CPW - 1)
                def _(q=q, off=off, ln=ln):
                    c1 = c + 1
                    pltpu.async_copy(
                        tab_hbm.at[c1 // _D, c1 % _D, pl.ds(off, ln)],
                        bufs[q], sems[q])

            pltpu.sync_copy(row_v, out_hbm.at[c])
            return f

        lax.fori_loop(0, _CPW, pair, jnp.int32(-1))

    return col_kernel


_KERNEL = _make_kernel()


def kernel(inputs, tables):
    tab_t = tables.transpose(0, 2, 1)
    idx_t = inputs.astype(jnp.int32).T
    out_t = _KERNEL(idx_t, tab_t)
    return out_t.T.reshape(_B, 1, _NF * _D)


# speedup vs baseline: 1.1691x; 1.1691x over previous
import functools

import jax
import jax.numpy as jnp
from jax import lax
from jax.experimental import pallas as pl
from jax.experimental.pallas import tpu as pltpu
from jax.experimental.pallas import tpu_sc as plsc

_NF = 26
_V = 100000
_D = 32
_B = 4096

_Q = 33280
_SPLIT = ((0, _Q), (_Q, _Q), (2 * _Q, _V - 2 * _Q))

_INFO = plsc.get_sparse_core_info()
_NC = _INFO.num_cores
_NS = _INFO.num_subcores
_NW = _NC * _NS
_CPW = _NF * _D // _NW
_UNROLL = 8


def _make_kernel():
    mesh = plsc.VectorSubcoreMesh(core_axis_name="c", subcore_axis_name="s")

    @functools.partial(
        pl.kernel,
        mesh=mesh,
        out_type=jax.ShapeDtypeStruct((_NF * _D, _B), jnp.float32),
        scratch_types=[
            pltpu.VMEM((_SPLIT[0][1],), jnp.float32),
            pltpu.VMEM((_SPLIT[1][1],), jnp.float32),
            pltpu.VMEM((_SPLIT[2][1],), jnp.float32),
            pltpu.VMEM((_B,), jnp.int32),
            pltpu.VMEM((_B,), jnp.float32),
            pltpu.SemaphoreType.DMA,
            pltpu.SemaphoreType.DMA,
            pltpu.SemaphoreType.DMA,
        ],
        compiler_params=pltpu.CompilerParams(use_tc_tiling_on_sc=True,
                                             needs_layout_passes=False),
    )
    def col_kernel(idx_hbm, tab_hbm, out_hbm, q0, q1, q2, idx_v, row_v,
                   s0, s1, s2):
        bufs = (q0, q1, q2)
        sems = (s0, s1, s2)
        w = lax.axis_index("s") * _NC + lax.axis_index("c")
        c0 = w * _CPW
        lane = lax.iota(jnp.int32, 16)

        for q, (off, ln) in enumerate(_SPLIT):
            pltpu.async_copy(tab_hbm.at[c0 // _D, c0 % _D, pl.ds(off, ln)],
                             bufs[q], sems[q])

        def pair(j, f_prev):
            c = c0 + j
            f = c // _D
            d = c % _D

            @pl.when(f != f_prev)
            def _():
                pltpu.sync_copy(idx_hbm.at[f], idx_v)

            for q, (off, ln) in enumerate(_SPLIT):
                pltpu.make_async_copy(tab_hbm.at[f, d, pl.ds(off, ln)],
                                      bufs[q], sems[q]).wait()
                buf = bufs[q]

                def gat(i, c2, q=q, off=off, ln=ln, buf=buf):
                    for u in range(_UNROLL):
                        sl = pl.ds((i * _UNROLL + u) * 16, 16)
                        ix = idx_v[sl] - off
                        m = lax.bitcast_convert_type(ix, jnp.uint32) < ln
                        g = plsc.load_gather(buf, [ix], mask=m)
                        if q == 0:
                            row_v[sl] = jnp.where(m, g, 0.0)
                        else:
                            row_v[sl] = jnp.where(m, g, row_v[sl])
                    return c2

                lax.fori_loop(0, _B // (16 * _UNROLL), gat, 0)

                @pl.when(j < _CPW - 1)
                def _(q=q, off=off, ln=ln):
                    c1 = c + 1
                    pltpu.async_copy(
                        tab_hbm.at[c1 // _D, c1 % _D, pl.ds(off, ln)],
                        bufs[q], sems[q])

            pltpu.sync_copy(row_v, out_hbm.at[c])
            return f

        lax.fori_loop(0, _CPW, pair, jnp.int32(-1))

    return col_kernel


_KERNEL = _make_kernel()


def kernel(inputs, tables):
    tab_t = tables.transpose(0, 2, 1)
    idx_t = inputs.astype(jnp.int32).T
    out_t = _KERNEL(idx_t, tab_t)
    return out_t.T.reshape(_B, 1, _NF * _D)


# speedup vs baseline: 1.1734x; 1.0037x over previous
import functools

import jax
import jax.numpy as jnp
from jax import lax
from jax.experimental import pallas as pl
from jax.experimental.pallas import tpu as pltpu
from jax.experimental.pallas import tpu_sc as plsc

_NF = 26
_V = 100000
_D = 32
_B = 4096

_Q = 33280
_SPLIT = ((0, _Q), (_Q, _Q), (2 * _Q, _V - 2 * _Q))

_INFO = plsc.get_sparse_core_info()
_NC = _INFO.num_cores
_NS = _INFO.num_subcores
_NW = _NC * _NS
_CPW = _NF * _D // _NW
_UNROLL = 8


def _make_kernel():
    mesh = plsc.VectorSubcoreMesh(core_axis_name="c", subcore_axis_name="s")

    @functools.partial(
        pl.kernel,
        mesh=mesh,
        out_type=jax.ShapeDtypeStruct((_NF * _D, _B), jnp.float32),
        scratch_types=[
            pltpu.VMEM((_SPLIT[0][1],), jnp.float32),
            pltpu.VMEM((_SPLIT[1][1],), jnp.float32),
            pltpu.VMEM((_SPLIT[2][1],), jnp.float32),
            pltpu.VMEM((_B,), jnp.int32),
            pltpu.VMEM((_B,), jnp.float32),
            pltpu.SemaphoreType.DMA,
            pltpu.SemaphoreType.DMA,
            pltpu.SemaphoreType.DMA,
        ],
        compiler_params=pltpu.CompilerParams(use_tc_tiling_on_sc=True,
                                             needs_layout_passes=False),
    )
    def col_kernel(idx_hbm, tab_hbm, out_hbm, q0, q1, q2, idx_v, row_v,
                   s0, s1, s2):
        bufs = (q0, q1, q2)
        sems = (s0, s1, s2)
        w = lax.axis_index("s") * _NC + lax.axis_index("c")
        c0 = w * _CPW

        for q, (off, ln) in enumerate(_SPLIT):
            pltpu.async_copy(tab_hbm.at[c0 // _D, c0 % _D, pl.ds(off, ln)],
                             bufs[q], sems[q])

        def pair(j, f_prev):
            c = c0 + j
            f = c // _D
            d = c % _D

            @pl.when(f != f_prev)
            def _():
                pltpu.sync_copy(idx_hbm.at[f], idx_v)

            for q, (off, ln) in enumerate(_SPLIT):
                pltpu.make_async_copy(tab_hbm.at[f, d, pl.ds(off, ln)],
                                      bufs[q], sems[q]).wait()
                buf = bufs[q]

                def gat(i, c2, q=q, off=off, ln=ln, buf=buf):
                    for u in range(_UNROLL):
                        sl = pl.ds((i * _UNROLL + u) * 16, 16)
                        ix = idx_v[sl] - off
                        m = lax.bitcast_convert_type(ix, jnp.uint32) < ln
                        g = plsc.load_gather(buf, [ix], mask=m)
                        if q == 0:
                            row_v[sl] = jnp.where(m, g, 0.0)
                        else:
                            row_v[sl] = jnp.where(m, g, row_v[sl])
                    return c2

                lax.fori_loop(0, _B // (16 * _UNROLL), gat, 0)

                @pl.when(j < _CPW - 1)
                def _(q=q, off=off, ln=ln):
                    c1 = c + 1
                    pltpu.async_copy(
                        tab_hbm.at[c1 // _D, c1 % _D, pl.ds(off, ln)],
                        bufs[q], sems[q])

            pltpu.sync_copy(row_v, out_hbm.at[c])
            return f

        lax.fori_loop(0, _CPW, pair, jnp.int32(-1))

    return col_kernel


_KERNEL = _make_kernel()


def kernel(inputs, tables):
    tab_t = tables.transpose(0, 2, 1)
    idx_t = inputs.astype(jnp.int32).T
    out_t = _KERNEL(idx_t, tab_t)
    return out_t.T.reshape(_B, 1, _NF * _D)
